# baseline (speedup 1.0000x reference)
import functools

import jax
import jax.numpy as jnp
from jax import lax
from jax.experimental import pallas as pl
from jax.experimental.pallas import tpu as pltpu
from jax.experimental.pallas import tpu_sc as plsc

D = 1024
CH = 1024
E = 8
HID = 4096
TOPK = 2
T = 4096
M = 256
NT = T * TOPK // M + E
P = NT * M
HC = 512
NH = HID // HC

_SC_NC = 2
_SC_NS = 16
_NW = _SC_NC * _SC_NS


def _gate_kernel(x_ref, sw_ref, sel_ref, w_ref):
    logits = lax.dot_general(
        x_ref[...].astype(jnp.bfloat16), sw_ref[...].astype(jnp.bfloat16),
        (((1,), (1,)), ((), ())), preferred_element_type=jnp.float32)
    idx = lax.broadcasted_iota(jnp.int32, logits.shape, 1)
    m1 = jnp.max(logits, axis=1, keepdims=True)
    i1 = jnp.min(jnp.where(logits == m1, idx, E), axis=1, keepdims=True)
    l2 = jnp.where(idx == i1, -jnp.inf, logits)
    m2 = jnp.max(l2, axis=1, keepdims=True)
    i2 = jnp.min(jnp.where(l2 == m2, idx, E), axis=1, keepdims=True)
    w1 = 1.0 / (1.0 + jnp.exp(m2 - m1))
    col = lax.broadcasted_iota(jnp.int32, (x_ref.shape[0], TOPK), 1)
    sel_ref[...] = jnp.where(col == 0, i1, i2)
    w_ref[...] = jnp.where(col == 0, w1, 1.0 - w1)


def _ffn_kernel(te_ref, xg_ref, w_ref, f1_ref, f3_ref, f2_ref, y_ref, acc_ref):
    h = pl.program_id(1)
    xb = xg_ref[...]
    h1 = lax.dot_general(xb, f1_ref[0], (((1,), (1,)), ((), ())),
                         preferred_element_type=jnp.float32)
    h3 = lax.dot_general(xb, f3_ref[0], (((1,), (1,)), ((), ())),
                         preferred_element_type=jnp.float32)
    g = jax.nn.silu(h1) * h3 * w_ref[...]
    part = lax.dot_general(g.astype(jnp.bfloat16), f2_ref[0],
                           (((1,), (1,)), ((), ())),
                           preferred_element_type=jnp.float32)

    @pl.when(h == 0)
    def _():
        acc_ref[...] = part

    @pl.when(h > 0)
    def _():
        acc_ref[...] += part

    @pl.when(h == NH - 1)
    def _():
        y_ref[...] = acc_ref[...].astype(jnp.bfloat16)


def _add_kernel(a_ref, b_ref, o_ref):
    o_ref[...] = a_ref[...].astype(jnp.float32) + b_ref[...].astype(jnp.float32)


_RING = 3
_CHUNK = 64
_DI = D // 2


def _to_i32(a):
    return lax.bitcast_convert_type(
        a.reshape(a.shape[:-1] + (a.shape[-1] // 2, 2)), jnp.int32)


def _to_bf16(a):
    b = lax.bitcast_convert_type(a, jnp.bfloat16)
    return b.reshape(b.shape[:-2] + (b.shape[-2] * 2,))


def _make_sc_gather(n_rows):
    b_per_w = n_rows // _NW
    n_ch = b_per_w // _CHUNK
    mesh = plsc.VectorSubcoreMesh(core_axis_name="c", subcore_axis_name="s")

    def body(src_hbm, idx_hbm, out_hbm, idx_v, *bufsem):
        bufs = bufsem[:_RING]
        sems = bufsem[_RING:]
        wid = lax.axis_index("s") * _SC_NC + lax.axis_index("c")
        base = wid * b_per_w
        pltpu.sync_copy(idx_hbm.at[pl.ds(base, b_per_w)], idx_v)
        cps = [None] * _RING
        for c in range(n_ch):
            p = c % _RING
            if cps[p] is not None:
                pcp, pc = cps[p]
                pcp.wait()
                pltpu.sync_copy(bufs[p],
                                out_hbm.at[pl.ds(base + pc * _CHUNK, _CHUNK)])
            cps[p] = (pltpu.async_copy(
                src_hbm.at[idx_v.at[pl.ds(c * _CHUNK, _CHUNK)]],
                bufs[p], sems[p]), c)
        for p in range(_RING):
            if cps[p] is not None:
                pcp, pc = cps[p]
                pcp.wait()
                pltpu.sync_copy(bufs[p],
                                out_hbm.at[pl.ds(base + pc * _CHUNK, _CHUNK)])

    return functools.partial(
        pl.kernel, body, mesh=mesh,
        out_type=jax.ShapeDtypeStruct((n_rows, _DI), jnp.int32),
        scratch_types=[pltpu.VMEM((b_per_w,), jnp.int32)]
        + [pltpu.VMEM((_CHUNK, _DI), jnp.int32) for _ in range(_RING)]
        + [pltpu.SemaphoreType.DMA for _ in range(_RING)])()


def kernel(x, switch_w, switch_b, fn1_w, fn1_b, fn2_w, fn2_b, fn3_w, fn3_b):
    B, N, d = x.shape
    xf = x.reshape(-1, d)

    sel, w2 = pl.pallas_call(
        _gate_kernel,
        in_specs=[pl.BlockSpec((T, D), lambda: (0, 0)),
                  pl.BlockSpec((E, D), lambda: (0, 0))],
        out_specs=[pl.BlockSpec((T, TOPK), lambda: (0, 0)),
                   pl.BlockSpec((T, TOPK), lambda: (0, 0))],
        out_shape=[jax.ShapeDtypeStruct((T, TOPK), jnp.int32),
                   jax.ShapeDtypeStruct((T, TOPK), jnp.float32)],
    )(xf, switch_w)

    e_flat = sel.reshape(-1)
    w_flat = w2.reshape(-1)
    tok_flat = jnp.arange(T * TOPK, dtype=jnp.int32) // TOPK
    onehot = (e_flat[:, None] == jnp.arange(E, dtype=jnp.int32)).astype(jnp.int32)
    counts = onehot.sum(axis=0)
    rank = jnp.take_along_axis(jnp.cumsum(onehot, axis=0) - onehot,
                               e_flat[:, None], axis=1)[:, 0]
    pad_cnt = ((counts + M - 1) // M) * M
    cum_pad = jnp.cumsum(pad_cnt)
    pad_start = cum_pad - pad_cnt
    pos = (pad_start[e_flat] + rank).astype(jnp.int32)
    packed = jnp.stack([tok_flat, lax.bitcast_convert_type(w_flat, jnp.int32)],
                       axis=1)
    slots = jnp.zeros((P, 2), jnp.int32).at[pos].set(packed)
    tok_slot = slots[:, 0]
    w_slot = lax.bitcast_convert_type(slots[:, 1], jnp.float32)
    tile_start = jnp.arange(NT, dtype=jnp.int32) * M
    tile_expert = jnp.minimum(
        (tile_start[:, None] >= cum_pad[None, :]).sum(axis=1), E - 1
    ).astype(jnp.int32)

    xb16 = xf.astype(jnp.bfloat16)
    xg = _to_bf16(_make_sc_gather(P)(_to_i32(xb16), tok_slot))

    f1b = fn1_w.astype(jnp.bfloat16)
    f3b = fn3_w.astype(jnp.bfloat16)
    f2b = fn2_w.astype(jnp.bfloat16)
    y = pl.pallas_call(
        _ffn_kernel,
        grid_spec=pltpu.PrefetchScalarGridSpec(
            num_scalar_prefetch=1,
            grid=(NT, NH),
            in_specs=[
                pl.BlockSpec((M, D), lambda t, h, te: (t, 0)),
                pl.BlockSpec((M, 1), lambda t, h, te: (t, 0)),
                pl.BlockSpec((1, HC, D), lambda t, h, te: (te[t], h, 0)),
                pl.BlockSpec((1, HC, D), lambda t, h, te: (te[t], h, 0)),
                pl.BlockSpec((1, CH, HC), lambda t, h, te: (te[t], 0, h)),
            ],
            out_specs=pl.BlockSpec((M, CH), lambda t, h, te: (t, 0)),
            scratch_shapes=[pltpu.VMEM((M, CH), jnp.float32)],
        ),
        out_shape=jax.ShapeDtypeStruct((P, CH), jnp.bfloat16),
        compiler_params=pltpu.CompilerParams(
            dimension_semantics=("arbitrary", "arbitrary")),
    )(tile_expert, xg, w_slot[:, None], f1b, f3b, f2b)

    slot_ab = pos.reshape(T, TOPK).T.reshape(-1)
    y_ab = _to_bf16(_make_sc_gather(2 * T)(_to_i32(y), slot_ab))
    out = pl.pallas_call(
        _add_kernel,
        grid=(4,),
        in_specs=[pl.BlockSpec((T // 4, CH), lambda i: (i, 0)),
                  pl.BlockSpec((T // 4, CH), lambda i: (i + 4, 0))],
        out_specs=pl.BlockSpec((T // 4, CH), lambda i: (i, 0)),
        out_shape=jax.ShapeDtypeStruct((T, CH), jnp.float32),
    )(y_ab, y_ab)
    return out.reshape(B, N, CH)


# speedup vs baseline: 1.6748x; 1.6748x over previous
import functools

import jax
import jax.numpy as jnp
from jax import lax
from jax.experimental import pallas as pl
from jax.experimental.pallas import tpu as pltpu
from jax.experimental.pallas import tpu_sc as plsc

D = 1024
CH = 1024
E = 8
HID = 4096
TOPK = 2
T = 4096
M = 256
NT = T * TOPK // M + E
P = NT * M
HC = 512
NH = HID // HC
DI = D // 2

_SC_NC = 2
_SC_NS = 16
_NW = _SC_NC * _SC_NS
_RING = 3
_CHUNK = 64


def _pack(lo_f32, hi_f32):
    lo = lax.bitcast_convert_type(lo_f32.astype(jnp.bfloat16), jnp.int16)
    hi = lax.bitcast_convert_type(hi_f32.astype(jnp.bfloat16), jnp.int16)
    return (hi.astype(jnp.int32) << 16) | (lo.astype(jnp.int32) & 0xFFFF)


def _unpack(v):
    lo = lax.bitcast_convert_type((v & 0xFFFF).astype(jnp.int16), jnp.bfloat16)
    hi = lax.bitcast_convert_type((v >> 16).astype(jnp.int16), jnp.bfloat16)
    return jnp.concatenate([lo, hi], axis=1)


def _gate_kernel(x_ref, sw_ref, sel_ref, w_ref, xp_ref):
    xv = x_ref[...]
    logits = lax.dot_general(
        xv.astype(jnp.bfloat16), sw_ref[...].astype(jnp.bfloat16),
        (((1,), (1,)), ((), ())), preferred_element_type=jnp.float32)
    idx = lax.broadcasted_iota(jnp.int32, logits.shape, 1)
    m1 = jnp.max(logits, axis=1, keepdims=True)
    i1 = jnp.min(jnp.where(logits == m1, idx, E), axis=1, keepdims=True)
    l2 = jnp.where(idx == i1, -jnp.inf, logits)
    m2 = jnp.max(l2, axis=1, keepdims=True)
    i2 = jnp.min(jnp.where(l2 == m2, idx, E), axis=1, keepdims=True)
    w1 = 1.0 / (1.0 + jnp.exp(m2 - m1))
    col = lax.broadcasted_iota(jnp.int32, (x_ref.shape[0], TOPK), 1)
    sel_ref[...] = jnp.where(col == 0, i1, i2)
    w_ref[...] = jnp.where(col == 0, w1, 1.0 - w1)
    xp_ref[...] = _pack(xv[:, :DI], xv[:, DI:])


def _ffn_kernel(te_ref, xg_ref, w_ref, f1_ref, f3_ref, f2_ref, y_ref, acc_ref):
    h = pl.program_id(1)
    xb = _unpack(xg_ref[...])
    h1 = lax.dot_general(xb, f1_ref[0], (((1,), (1,)), ((), ())),
                         preferred_element_type=jnp.float32)
    h3 = lax.dot_general(xb, f3_ref[0], (((1,), (1,)), ((), ())),
                         preferred_element_type=jnp.float32)
    g = jax.nn.silu(h1) * h3 * w_ref[...]
    part = lax.dot_general(g.astype(jnp.bfloat16), f2_ref[0],
                           (((1,), (1,)), ((), ())),
                           preferred_element_type=jnp.float32)

    @pl.when(h == 0)
    def _():
        acc_ref[...] = part

    @pl.when(h > 0)
    def _():
        acc_ref[...] += part

    @pl.when(h == NH - 1)
    def _():
        acc = acc_ref[...]
        y_ref[...] = _pack(acc[:, :DI], acc[:, DI:])


def _add_kernel(a_ref, b_ref, o_ref):
    s = (_unpack(a_ref[...]).astype(jnp.float32)
         + _unpack(b_ref[...]).astype(jnp.float32))
    o_ref[...] = s


def _make_sc_gather(n_rows):
    b_per_w = n_rows // _NW
    n_ch = b_per_w // _CHUNK
    mesh = plsc.VectorSubcoreMesh(core_axis_name="c", subcore_axis_name="s")

    def body(src_hbm, idx_hbm, out_hbm, idx_v, *bufsem):
        bufs = bufsem[:_RING]
        sems = bufsem[_RING:]
        wid = lax.axis_index("s") * _SC_NC + lax.axis_index("c")
        base = wid * b_per_w
        pltpu.sync_copy(idx_hbm.at[pl.ds(base, b_per_w)], idx_v)
        cps = [None] * _RING
        for c in range(n_ch):
            p = c % _RING
            if cps[p] is not None:
                pcp, pc = cps[p]
                pcp.wait()
                pltpu.sync_copy(bufs[p],
                                out_hbm.at[pl.ds(base + pc * _CHUNK, _CHUNK)])
            cps[p] = (pltpu.async_copy(
                src_hbm.at[idx_v.at[pl.ds(c * _CHUNK, _CHUNK)]],
                bufs[p], sems[p]), c)
        for p in range(_RING):
            if cps[p] is not None:
                pcp, pc = cps[p]
                pcp.wait()
                pltpu.sync_copy(bufs[p],
                                out_hbm.at[pl.ds(base + pc * _CHUNK, _CHUNK)])

    return functools.partial(
        pl.kernel, body, mesh=mesh,
        out_type=jax.ShapeDtypeStruct((n_rows, DI), jnp.int32),
        scratch_types=[pltpu.VMEM((b_per_w,), jnp.int32)]
        + [pltpu.VMEM((_CHUNK, DI), jnp.int32) for _ in range(_RING)]
        + [pltpu.SemaphoreType.DMA for _ in range(_RING)])()


def kernel(x, switch_w, switch_b, fn1_w, fn1_b, fn2_w, fn2_b, fn3_w, fn3_b):
    B, N, d = x.shape
    xf = x.reshape(-1, d)

    sel, w2, xpack = pl.pallas_call(
        _gate_kernel,
        in_specs=[pl.BlockSpec((T, D), lambda: (0, 0)),
                  pl.BlockSpec((E, D), lambda: (0, 0))],
        out_specs=[pl.BlockSpec((T, TOPK), lambda: (0, 0)),
                   pl.BlockSpec((T, TOPK), lambda: (0, 0)),
                   pl.BlockSpec((T, DI), lambda: (0, 0))],
        out_shape=[jax.ShapeDtypeStruct((T, TOPK), jnp.int32),
                   jax.ShapeDtypeStruct((T, TOPK), jnp.float32),
                   jax.ShapeDtypeStruct((T, DI), jnp.int32)],
    )(xf, switch_w)

    e_flat = sel.reshape(-1)
    w_flat = w2.reshape(-1)
    tok_flat = jnp.arange(T * TOPK, dtype=jnp.int32) // TOPK
    onehot = (e_flat[:, None] == jnp.arange(E, dtype=jnp.int32)).astype(jnp.int32)
    counts = onehot.sum(axis=0)
    rank = jnp.take_along_axis(jnp.cumsum(onehot, axis=0) - onehot,
                               e_flat[:, None], axis=1)[:, 0]
    pad_cnt = ((counts + M - 1) // M) * M
    cum_pad = jnp.cumsum(pad_cnt)
    pad_start = cum_pad - pad_cnt
    pos = (pad_start[e_flat] + rank).astype(jnp.int32)
    packed = jnp.stack([tok_flat, lax.bitcast_convert_type(w_flat, jnp.int32)],
                       axis=1)
    slots = jnp.zeros((P, 2), jnp.int32).at[pos].set(packed)
    tok_slot = slots[:, 0]
    w_slot = lax.bitcast_convert_type(slots[:, 1], jnp.float32)
    tile_start = jnp.arange(NT, dtype=jnp.int32) * M
    tile_expert = jnp.minimum(
        (tile_start[:, None] >= cum_pad[None, :]).sum(axis=1), E - 1
    ).astype(jnp.int32)

    xg = _make_sc_gather(P)(xpack, tok_slot)

    f1b = fn1_w.astype(jnp.bfloat16)
    f3b = fn3_w.astype(jnp.bfloat16)
    f2b = fn2_w.astype(jnp.bfloat16)
    y = pl.pallas_call(
        _ffn_kernel,
        grid_spec=pltpu.PrefetchScalarGridSpec(
            num_scalar_prefetch=1,
            grid=(NT, NH),
            in_specs=[
                pl.BlockSpec((M, DI), lambda t, h, te: (t, 0)),
                pl.BlockSpec((M, 1), lambda t, h, te: (t, 0)),
                pl.BlockSpec((1, HC, D), lambda t, h, te: (te[t], h, 0)),
                pl.BlockSpec((1, HC, D), lambda t, h, te: (te[t], h, 0)),
                pl.BlockSpec((1, CH, HC), lambda t, h, te: (te[t], 0, h)),
            ],
            out_specs=pl.BlockSpec((M, DI), lambda t, h, te: (t, 0)),
            scratch_shapes=[pltpu.VMEM((M, CH), jnp.float32)],
        ),
        out_shape=jax.ShapeDtypeStruct((P, DI), jnp.int32),
        compiler_params=pltpu.CompilerParams(
            dimension_semantics=("arbitrary", "arbitrary")),
    )(tile_expert, xg, w_slot[:, None], f1b, f3b, f2b)

    slot_ab = pos.reshape(T, TOPK).T.reshape(-1)
    y_ab = _make_sc_gather(2 * T)(y, slot_ab)
    out = pl.pallas_call(
        _add_kernel,
        grid=(4,),
        in_specs=[pl.BlockSpec((T // 4, DI), lambda i: (i, 0)),
                  pl.BlockSpec((T // 4, DI), lambda i: (i + 4, 0))],
        out_specs=pl.BlockSpec((T // 4, CH), lambda i: (i, 0)),
        out_shape=jax.ShapeDtypeStruct((T, CH), jnp.float32),
    )(y_ab, y_ab)
    return out.reshape(B, N, CH)


# speedup vs baseline: 1.9178x; 1.1451x over previous
import functools

import jax
import jax.numpy as jnp
from jax import lax
from jax.experimental import pallas as pl
from jax.experimental.pallas import tpu as pltpu
from jax.experimental.pallas import tpu_sc as plsc

D = 1024
CH = 1024
E = 8
HID = 4096
TOPK = 2
T = 4096
M = 256
NT = T * TOPK // M + E
P = NT * M
HC = 1024
NH = HID // HC
DI = D // 2

_SC_NC = 2
_SC_NS = 16
_NW = _SC_NC * _SC_NS
_RING = 3
_CHUNK = 64


def _pack(lo_f32, hi_f32):
    lo = lax.bitcast_convert_type(lo_f32.astype(jnp.bfloat16), jnp.int16)
    hi = lax.bitcast_convert_type(hi_f32.astype(jnp.bfloat16), jnp.int16)
    return (hi.astype(jnp.int32) << 16) | (lo.astype(jnp.int32) & 0xFFFF)


def _unpack(v):
    lo = lax.bitcast_convert_type((v & 0xFFFF).astype(jnp.int16), jnp.bfloat16)
    hi = lax.bitcast_convert_type((v >> 16).astype(jnp.int16), jnp.bfloat16)
    return jnp.concatenate([lo, hi], axis=1)


def _gate_kernel(x_ref, sw_ref, sel_ref, w_ref, xp_ref):
    xv = x_ref[...]
    logits = lax.dot_general(
        xv.astype(jnp.bfloat16), sw_ref[...].astype(jnp.bfloat16),
        (((1,), (1,)), ((), ())), preferred_element_type=jnp.float32)
    idx = lax.broadcasted_iota(jnp.int32, logits.shape, 1)
    m1 = jnp.max(logits, axis=1, keepdims=True)
    i1 = jnp.min(jnp.where(logits == m1, idx, E), axis=1, keepdims=True)
    l2 = jnp.where(idx == i1, -jnp.inf, logits)
    m2 = jnp.max(l2, axis=1, keepdims=True)
    i2 = jnp.min(jnp.where(l2 == m2, idx, E), axis=1, keepdims=True)
    w1 = 1.0 / (1.0 + jnp.exp(m2 - m1))
    col = lax.broadcasted_iota(jnp.int32, (x_ref.shape[0], TOPK), 1)
    sel_ref[...] = jnp.where(col == 0, i1, i2)
    w_ref[...] = jnp.where(col == 0, w1, 1.0 - w1)
    xp_ref[...] = _pack(xv[:, :DI], xv[:, DI:])


def _ffn_kernel(te_ref, xg_ref, w_ref, f1_ref, f3_ref, f2_ref, y_ref, acc_ref):
    h = pl.program_id(1)
    xb = _unpack(xg_ref[...])
    h1 = lax.dot_general(xb, f1_ref[0], (((1,), (1,)), ((), ())),
                         preferred_element_type=jnp.float32)
    h3 = lax.dot_general(xb, f3_ref[0], (((1,), (1,)), ((), ())),
                         preferred_element_type=jnp.float32)
    g = jax.nn.silu(h1) * h3 * w_ref[...]
    part = lax.dot_general(g.astype(jnp.bfloat16), f2_ref[0],
                           (((1,), (1,)), ((), ())),
                           preferred_element_type=jnp.float32)

    @pl.when(h == 0)
    def _():
        acc_ref[...] = part

    @pl.when(h > 0)
    def _():
        acc_ref[...] += part

    @pl.when(h == NH - 1)
    def _():
        acc = acc_ref[...]
        y_ref[...] = _pack(acc[:, :DI], acc[:, DI:])


def _add_kernel(a_ref, b_ref, o_ref):
    s = (_unpack(a_ref[...]).astype(jnp.float32)
         + _unpack(b_ref[...]).astype(jnp.float32))
    o_ref[...] = s


def _make_sc_gather(n_rows):
    b_per_w = n_rows // _NW
    n_ch = b_per_w // _CHUNK
    mesh = plsc.VectorSubcoreMesh(core_axis_name="c", subcore_axis_name="s")

    def body(src_hbm, idx_hbm, out_hbm, idx_v, *bufsem):
        bufs = bufsem[:_RING]
        gsems = bufsem[_RING:2 * _RING]
        ssems = bufsem[2 * _RING:]
        wid = lax.axis_index("s") * _SC_NC + lax.axis_index("c")
        base = wid * b_per_w
        pltpu.sync_copy(idx_hbm.at[pl.ds(base, b_per_w)], idx_v)
        gcp = [None] * _RING
        scp = [None] * _RING
        for c in range(n_ch):
            p = c % _RING
            if scp[p] is not None:
                scp[p].wait()
            gcp[p] = pltpu.async_copy(
                src_hbm.at[idx_v.at[pl.ds(c * _CHUNK, _CHUNK)]],
                bufs[p], gsems[p])
            if c > 0:
                q = (c - 1) % _RING
                gcp[q].wait()
                scp[q] = pltpu.async_copy(
                    bufs[q], out_hbm.at[pl.ds(base + (c - 1) * _CHUNK, _CHUNK)],
                    ssems[q])
        q = (n_ch - 1) % _RING
        gcp[q].wait()
        scp[q] = pltpu.async_copy(
            bufs[q], out_hbm.at[pl.ds(base + (n_ch - 1) * _CHUNK, _CHUNK)],
            ssems[q])
        for p in range(_RING):
            if scp[p] is not None:
                scp[p].wait()

    return functools.partial(
        pl.kernel, body, mesh=mesh,
        out_type=jax.ShapeDtypeStruct((n_rows, DI), jnp.int32),
        scratch_types=[pltpu.VMEM((b_per_w,), jnp.int32)]
        + [pltpu.VMEM((_CHUNK, DI), jnp.int32) for _ in range(_RING)]
        + [pltpu.SemaphoreType.DMA for _ in range(2 * _RING)])()


def kernel(x, switch_w, switch_b, fn1_w, fn1_b, fn2_w, fn2_b, fn3_w, fn3_b):
    B, N, d = x.shape
    xf = x.reshape(-1, d)

    sel, w2, xpack = pl.pallas_call(
        _gate_kernel,
        in_specs=[pl.BlockSpec((T, D), lambda: (0, 0)),
                  pl.BlockSpec((E, D), lambda: (0, 0))],
        out_specs=[pl.BlockSpec((T, TOPK), lambda: (0, 0)),
                   pl.BlockSpec((T, TOPK), lambda: (0, 0)),
                   pl.BlockSpec((T, DI), lambda: (0, 0))],
        out_shape=[jax.ShapeDtypeStruct((T, TOPK), jnp.int32),
                   jax.ShapeDtypeStruct((T, TOPK), jnp.float32),
                   jax.ShapeDtypeStruct((T, DI), jnp.int32)],
    )(xf, switch_w)

    e_flat = sel.reshape(-1)
    w_flat = w2.reshape(-1)
    tok_flat = jnp.arange(T * TOPK, dtype=jnp.int32) // TOPK
    onehot = (e_flat[:, None] == jnp.arange(E, dtype=jnp.int32)).astype(jnp.int32)
    counts = onehot.sum(axis=0)
    rank = jnp.take_along_axis(jnp.cumsum(onehot, axis=0) - onehot,
                               e_flat[:, None], axis=1)[:, 0]
    pad_cnt = ((counts + M - 1) // M) * M
    cum_pad = jnp.cumsum(pad_cnt)
    pad_start = cum_pad - pad_cnt
    pos = (pad_start[e_flat] + rank).astype(jnp.int32)
    packed = jnp.stack([tok_flat, lax.bitcast_convert_type(w_flat, jnp.int32)],
                       axis=1)
    slots = jnp.zeros((P, 2), jnp.int32).at[pos].set(packed)
    tok_slot = slots[:, 0]
    w_slot = lax.bitcast_convert_type(slots[:, 1], jnp.float32)
    tile_start = jnp.arange(NT, dtype=jnp.int32) * M
    tile_expert = jnp.minimum(
        (tile_start[:, None] >= cum_pad[None, :]).sum(axis=1), E - 1
    ).astype(jnp.int32)

    xg = _make_sc_gather(P)(xpack, tok_slot)

    f1b = fn1_w.astype(jnp.bfloat16)
    f3b = fn3_w.astype(jnp.bfloat16)
    f2b = fn2_w.astype(jnp.bfloat16)
    y = pl.pallas_call(
        _ffn_kernel,
        grid_spec=pltpu.PrefetchScalarGridSpec(
            num_scalar_prefetch=1,
            grid=(NT, NH),
            in_specs=[
                pl.BlockSpec((M, DI), lambda t, h, te: (t, 0)),
                pl.BlockSpec((M, 1), lambda t, h, te: (t, 0)),
                pl.BlockSpec((1, HC, D), lambda t, h, te: (te[t], h, 0)),
                pl.BlockSpec((1, HC, D), lambda t, h, te: (te[t], h, 0)),
                pl.BlockSpec((1, CH, HC), lambda t, h, te: (te[t], 0, h)),
            ],
            out_specs=pl.BlockSpec((M, DI), lambda t, h, te: (t, 0)),
            scratch_shapes=[pltpu.VMEM((M, CH), jnp.float32)],
        ),
        out_shape=jax.ShapeDtypeStruct((P, DI), jnp.int32),
        compiler_params=pltpu.CompilerParams(
            dimension_semantics=("arbitrary", "arbitrary")),
    )(tile_expert, xg, w_slot[:, None], f1b, f3b, f2b)

    slot_ab = pos.reshape(T, TOPK).T.reshape(-1)
    y_ab = _make_sc_gather(2 * T)(y, slot_ab)
    out = pl.pallas_call(
        _add_kernel,
        grid=(4,),
        in_specs=[pl.BlockSpec((T // 4, DI), lambda i: (i, 0)),
                  pl.BlockSpec((T // 4, DI), lambda i: (i + 4, 0))],
        out_specs=pl.BlockSpec((T // 4, CH), lambda i: (i, 0)),
        out_shape=jax.ShapeDtypeStruct((T, CH), jnp.float32),
    )(y_ab, y_ab)
    return out.reshape(B, N, CH)
